# baseline (device time: 46640 ns/iter reference)
import jax
import jax.numpy as jnp
from jax import lax
from jax.experimental import pallas as pl
from jax.experimental.pallas import tpu as pltpu

N_DEV = 4


def kernel(x, w_mat):
    m, _ = x.shape
    _, n = w_mat.shape
    m_blk = m // N_DEV

    def body(x_ref, w_ref, out_ref, comm_ref, send_sems, recv_sems):
        my = lax.axis_index("i")
        left = (my - 1) % N_DEV
        right = (my + 1) % N_DEV

        barrier_sem = pltpu.get_barrier_semaphore()
        for nbr in (left, right):
            pl.semaphore_signal(
                barrier_sem, inc=1,
                device_id=(nbr,), device_id_type=pl.DeviceIdType.MESH,
            )
        pl.semaphore_wait(barrier_sem, 2)

        def partial_chunk(c):
            rows = x_ref[pl.ds(c * m_blk, m_blk), :]
            return jnp.dot(rows, w_ref[:, :], preferred_element_type=jnp.float32)

        comm_ref[0, :, :] = partial_chunk((my - 1) % N_DEV)

        for s in range(N_DEV - 1):
            rdma = pltpu.make_async_remote_copy(
                src_ref=comm_ref.at[s],
                dst_ref=comm_ref.at[s + 1],
                send_sem=send_sems.at[s],
                recv_sem=recv_sems.at[s],
                device_id=(right,),
                device_id_type=pl.DeviceIdType.MESH,
            )
            rdma.start()
            rdma.wait()

            c = (my - 2 - s) % N_DEV
            acc = comm_ref[s + 1, :, :] + partial_chunk(c)
            if s < N_DEV - 2:
                comm_ref[s + 1, :, :] = acc
            else:
                out_ref[:, :] = jnp.maximum(acc, 0.0)

    return pl.pallas_call(
        body,
        out_shape=jax.ShapeDtypeStruct((m_blk, n), jnp.float32),
        in_specs=[
            pl.BlockSpec(memory_space=pltpu.VMEM),
            pl.BlockSpec(memory_space=pltpu.VMEM),
        ],
        out_specs=pl.BlockSpec(memory_space=pltpu.VMEM),
        scratch_shapes=[
            pltpu.VMEM((N_DEV, m_blk, n), jnp.float32),
            pltpu.SemaphoreType.DMA((N_DEV - 1,)),
            pltpu.SemaphoreType.DMA((N_DEV - 1,)),
        ],
        compiler_params=pltpu.CompilerParams(collective_id=0),
    )(x, w_mat)


# device time: 29267 ns/iter; 1.5936x vs baseline; 1.5936x over previous
import jax
import jax.numpy as jnp
from jax import lax
from jax.experimental import pallas as pl
from jax.experimental.pallas import tpu as pltpu

N_DEV = 4


def kernel(x, w_mat):
    m, _ = x.shape
    _, n = w_mat.shape
    m_blk = m // N_DEV
    n_half = n // 2

    def body(x_ref, w_ref, out_ref, cw_ref, ccw_ref,
             cw_send, cw_recv, ccw_send, ccw_recv):
        my = lax.axis_index("i")
        left = (my - 1) % N_DEV
        right = (my + 1) % N_DEV

        barrier_sem = pltpu.get_barrier_semaphore()
        for nbr in (left, right):
            pl.semaphore_signal(
                barrier_sem, inc=1,
                device_id=(nbr,), device_id_type=pl.DeviceIdType.MESH,
            )
        pl.semaphore_wait(barrier_sem, 2)

        def partial_half(c, lo):
            rows = x_ref[pl.ds(c * m_blk, m_blk), :]
            return jnp.dot(rows, w_ref[:, lo:lo + n_half],
                           preferred_element_type=jnp.float32)

        cw_ref[0, :, :] = partial_half((my - 1) % N_DEV, 0)
        ccw_ref[0, :, :] = partial_half((my + 1) % N_DEV, n_half)

        for s in range(N_DEV - 1):
            cw_rdma = pltpu.make_async_remote_copy(
                src_ref=cw_ref.at[s], dst_ref=cw_ref.at[s + 1],
                send_sem=cw_send.at[s], recv_sem=cw_recv.at[s],
                device_id=(right,), device_id_type=pl.DeviceIdType.MESH,
            )
            ccw_rdma = pltpu.make_async_remote_copy(
                src_ref=ccw_ref.at[s], dst_ref=ccw_ref.at[s + 1],
                send_sem=ccw_send.at[s], recv_sem=ccw_recv.at[s],
                device_id=(left,), device_id_type=pl.DeviceIdType.MESH,
            )
            cw_rdma.start()
            ccw_rdma.start()

            cw_add = partial_half((my - 2 - s) % N_DEV, 0)
            ccw_add = partial_half((my + 2 + s) % N_DEV, n_half)

            cw_rdma.wait()
            ccw_rdma.wait()

            if s < N_DEV - 2:
                cw_ref[s + 1, :, :] += cw_add
                ccw_ref[s + 1, :, :] += ccw_add
            else:
                out_ref[:, :n_half] = jnp.maximum(cw_ref[s + 1, :, :] + cw_add, 0.0)
                out_ref[:, n_half:] = jnp.maximum(ccw_ref[s + 1, :, :] + ccw_add, 0.0)

    return pl.pallas_call(
        body,
        out_shape=jax.ShapeDtypeStruct((m_blk, n), jnp.float32),
        in_specs=[
            pl.BlockSpec(memory_space=pltpu.VMEM),
            pl.BlockSpec(memory_space=pltpu.VMEM),
        ],
        out_specs=pl.BlockSpec(memory_space=pltpu.VMEM),
        scratch_shapes=[
            pltpu.VMEM((N_DEV, m_blk, n_half), jnp.float32),
            pltpu.VMEM((N_DEV, m_blk, n_half), jnp.float32),
            pltpu.SemaphoreType.DMA((N_DEV - 1,)),
            pltpu.SemaphoreType.DMA((N_DEV - 1,)),
            pltpu.SemaphoreType.DMA((N_DEV - 1,)),
            pltpu.SemaphoreType.DMA((N_DEV - 1,)),
        ],
        compiler_params=pltpu.CompilerParams(collective_id=0),
    )(x, w_mat)


# device time: 25328 ns/iter; 1.8414x vs baseline; 1.1555x over previous
import jax
import jax.numpy as jnp
from jax import lax
from jax.experimental import pallas as pl
from jax.experimental.pallas import tpu as pltpu

N_DEV = 4
N_SUB = 4


def kernel(x, w_mat):
    m, _ = x.shape
    _, n = w_mat.shape
    m_blk = m // N_DEV
    sub_n = n // N_SUB

    def body(x_ref, w_ref, out_ref, buf, send_sems, recv_sems):
        my = lax.axis_index("i")
        left = (my - 1) % N_DEV
        right = (my + 1) % N_DEV

        barrier_sem = pltpu.get_barrier_semaphore()
        for nbr in (left, right):
            pl.semaphore_signal(
                barrier_sem, inc=1,
                device_id=(nbr,), device_id_type=pl.DeviceIdType.MESH,
            )
        pl.semaphore_wait(barrier_sem, 2)

        def partial_sub(c, r):
            rows = x_ref[pl.ds(c * m_blk, m_blk), :]
            return jnp.dot(rows, w_ref[:, r * sub_n:(r + 1) * sub_n],
                           preferred_element_type=jnp.float32)

        def send_chunk(r, s):
            return (my - 1 - s) % N_DEV if r < N_SUB // 2 else (my + 1 + s) % N_DEV

        def recv_chunk(r, s):
            return (my - 2 - s) % N_DEV if r < N_SUB // 2 else (my + 2 + s) % N_DEV

        def make_rdma(r, s):
            return pltpu.make_async_remote_copy(
                src_ref=buf.at[r, s], dst_ref=buf.at[r, s + 1],
                send_sem=send_sems.at[r, s], recv_sem=recv_sems.at[r, s],
                device_id=(right if r < N_SUB // 2 else left,),
                device_id_type=pl.DeviceIdType.MESH,
            )

        for r in range(N_SUB):
            buf[r, 0, :, :] = partial_sub(send_chunk(r, 0), r)
            make_rdma(r, 0).start()

        for s in range(N_DEV - 1):
            for r in range(N_SUB):
                add = partial_sub(recv_chunk(r, s), r)
                make_rdma(r, s).wait_recv()
                if s < N_DEV - 2:
                    buf[r, s + 1, :, :] += add
                    make_rdma(r, s + 1).start()
                else:
                    out_ref[:, r * sub_n:(r + 1) * sub_n] = jnp.maximum(
                        buf[r, s + 1, :, :] + add, 0.0)

        for r in range(N_SUB):
            for s in range(N_DEV - 1):
                make_rdma(r, s).wait_send()

    return pl.pallas_call(
        body,
        out_shape=jax.ShapeDtypeStruct((m_blk, n), jnp.float32),
        in_specs=[
            pl.BlockSpec(memory_space=pltpu.VMEM),
            pl.BlockSpec(memory_space=pltpu.VMEM),
        ],
        out_specs=pl.BlockSpec(memory_space=pltpu.VMEM),
        scratch_shapes=[
            pltpu.VMEM((N_SUB, N_DEV, m_blk, sub_n), jnp.float32),
            pltpu.SemaphoreType.DMA((N_SUB, N_DEV - 1)),
            pltpu.SemaphoreType.DMA((N_SUB, N_DEV - 1)),
        ],
        compiler_params=pltpu.CompilerParams(collective_id=0),
    )(x, w_mat)
